# baseline (speedup 1.0000x reference)
import jax
import jax.numpy as jnp
from jax.experimental import pallas as pl
from jax.experimental.pallas import tpu as pltpu

NT = 128
CUT2 = 0.7 * 0.7
MAXD = 0.02
MAXV = 0.1

f32 = jnp.float32


def _dot(a, b):
    return jax.lax.dot_general(a, b, (((a.ndim - 1,), (0,)), ((), ())),
                               preferred_element_type=f32)


def _init_kernel(types_ref, feat_ref, embw_ref, vmw_ref, bias_ref, h0_ref):
    t = types_ref[...]
    nat = embw_ref.shape[0]
    onehot = (t == jax.lax.broadcasted_iota(jnp.int32, (NT, nat), 1)).astype(f32)
    h0 = _dot(onehot, embw_ref[...]) + _dot(feat_ref[...], vmw_ref[...])
    h0_ref[...] = h0 + bias_ref[0:1, :]


def _proj_kernel(h_ref, w1a_ref, w1b_ref, b1_ref, p1_ref, p2_ref):
    h = h_ref[...]
    p1_ref[...] = _dot(h, w1a_ref[...]) + b1_ref[0:1, :]
    p2_ref[...] = _dot(h, w1b_ref[...])


def _head_kernel(h_ref, hw_ref, hb_ref, out_ref):
    z = jnp.tanh(_dot(h_ref[...], hw_ref[...]) + hb_ref[0:1, :])
    out_ref[...] = z * hb_ref[1:2, :]


def _layer_kernel(cts_ref, ctn_ref,
                  x0r_ref, xr_ref, hr_ref, p1r_ref, br_ref,
                  xt0_ref, xt_ref, p2_ref, b3_ref,
                  w2_ref, cw1_ref, cw2_ref, nw1_ref, nw2_ref, vec_ref,
                  xout_ref, xtout_ref, hout_ref):
    r = pl.program_id(0)
    cstart = cts_ref[r]
    cnum = ctn_ref[r]
    x0i = x0r_ref[...]
    xi = xr_ref[...]
    hi = hr_ref[...]
    p1 = p1r_ref[...]
    bi = br_ref[...]
    wd = vec_ref[0:1, :]
    b2 = vec_ref[1:2, :]
    cb1 = vec_ref[2:3, :]
    cb2 = vec_ref[3:4, :]
    nb1 = vec_ref[4:5, :]
    nb2 = vec_ref[5:6, :]
    lng = vec_ref[6:7, :]
    lnb = vec_ref[7:8, :]
    w2 = w2_ref[...].astype(jnp.bfloat16)
    cw1 = cw1_ref[...].astype(jnp.bfloat16)
    cw2 = cw2_ref[...].astype(jnp.bfloat16)
    p1b = p1.astype(jnp.bfloat16)
    b2b = b2.astype(jnp.bfloat16)
    cb1b = cb1.astype(jnp.bfloat16)
    wdb = wd.astype(jnp.bfloat16)

    row_ids = r * NT + jax.lax.broadcasted_iota(jnp.int32, (NT, 1), 0)

    def body(c, carry):
        agg, s, t, cnt = carry
        ct = cstart + c
        xt0j = xt0_ref[ct]
        xtj = xt_ref[ct]
        p2j = p2_ref[ct]
        bj = b3_ref[ct]

        d0 = jnp.zeros((NT, NT), f32)
        d2 = jnp.zeros((NT, NT), f32)
        for d in range(3):
            f0 = x0i[:, d:d + 1] - xt0j[d:d + 1, :]
            d0 = d0 + f0 * f0
            f1 = xi[:, d:d + 1] - xtj[d:d + 1, :]
            d2 = d2 + f1 * f1
        col_ids = ct * NT + jax.lax.broadcasted_iota(jnp.int32, (1, NT), 1)
        mask = (d0 <= CUT2) & (bi == bj) & (row_ids != col_ids)
        mf = mask.astype(f32)

        bf = jnp.bfloat16
        d2b = d2.astype(bf)
        e3 = (p1b[:, None, :] + p2j.astype(bf)[None, :, :]
              + d2b[:, :, None] * wdb[None, :, :])
        ef1 = jax.nn.silu(e3).reshape(NT * NT, 128)
        ef2 = jax.nn.silu(_dot(ef1, w2).astype(bf) + b2b)
        cg1 = jax.nn.silu(_dot(ef2, cw1).astype(bf) + cb1b)
        g8 = _dot(cg1, cw2)
        gate = g8.reshape(NT, NT, 8).sum(axis=2) + cb2
        gm = gate * mf
        ef23 = ef2.reshape(NT, NT, 128)
        mf3 = mf.reshape(NT, 1, NT).astype(jnp.bfloat16)
        aggc = jax.lax.dot_general(
            mf3, ef23, (((2,), (1,)), ((0,), (0,))),
            preferred_element_type=f32)
        agg = agg + aggc.reshape(NT, 128)
        s = s + gm.sum(axis=1, keepdims=True)
        t = t + jax.lax.dot_general(gm, xtj, (((1,), (1,)), ((), ())),
                                    preferred_element_type=f32)
        cnt = cnt + mf.sum(axis=1, keepdims=True)
        return agg, s, t, cnt

    agg0 = (jnp.zeros((NT, 128), f32), jnp.zeros((NT, 1), f32),
            jnp.zeros((NT, 8), f32), jnp.zeros((NT, 1), f32))
    agg, s, t, cnt = jax.lax.fori_loop(0, cnum, body, agg0)

    cntc = jnp.maximum(cnt, 1.0)
    xn = xi + (xi * s - t) / cntc
    aggm = agg / cntc
    node_in = jnp.concatenate([hi, aggm], axis=1)
    d1 = jax.nn.silu(_dot(node_in, nw1_ref[...]) + nb1)
    hn = hi + _dot(d1, nw2_ref[...]) + nb2
    mu = jnp.mean(hn, axis=1, keepdims=True)
    var = jnp.mean((hn - mu) ** 2, axis=1, keepdims=True)
    hln = (hn - mu) * jax.lax.rsqrt(var + 1e-5) * lng + lnb
    hout_ref[...] = jax.nn.silu(hln)
    xout_ref[...] = xn
    xtout_ref[0] = xn.T


def _row_spec(w):
    return pl.BlockSpec((NT, w), lambda r, *_: (r, 0))


def _full_spec(shape):
    n = len(shape)
    return pl.BlockSpec(shape, lambda r, *_, _n=n: (0,) * _n)


def kernel(x_t, v_t, atom_types, masses, batch, params):
    N = x_t.shape[0]
    NG = 20
    i32 = jnp.int32
    nrt = N // NT + NG + 1
    NP = nrt * NT
    p = params

    b = batch.astype(i32)
    gidx = jnp.arange(NG, dtype=i32)
    gstart = jnp.searchsorted(b, gidx, side='left').astype(i32)
    gend = jnp.searchsorted(b, gidx, side='right').astype(i32)
    sizes = gend - gstart
    tcnt = (sizes + NT - 1) // NT
    prows = tcnt * NT
    pstart = (jnp.cumsum(prows) - prows).astype(i32)
    total_tiles = jnp.sum(tcnt)
    newpos = pstart[b] + (jnp.arange(N, dtype=i32) - gstart[b])

    x0 = jnp.zeros((NP, 8), f32).at[newpos, :3].set(x_t.astype(f32))
    feat = jnp.zeros((NP, 8), f32).at[newpos, :3].set(v_t.astype(f32)) \
                                  .at[newpos, 3].set(masses.astype(f32))
    types = jnp.full((NP, 1), -1, i32).at[newpos, 0].set(atom_types.astype(i32))
    bcol = jnp.full((NP,), -1, i32).at[newpos].set(b)
    brow = bcol.reshape(NP, 1)
    b3 = bcol.reshape(nrt, 1, NT)

    ar = jnp.arange(nrt, dtype=i32)
    tstartg = pstart // NT
    tg = jnp.clip(jnp.searchsorted(tstartg, ar, side='right').astype(i32) - 1,
                  0, NG - 1)
    ct_start = tstartg[tg]
    ct_num = jnp.where(ar < total_tiles, tcnt[tg], 0).astype(i32)

    nW = p['node_W'].astype(f32)
    embw = jnp.zeros((104, 128), f32).at[:100].set(p['atom_emb'].astype(f32) @ nW[:64])
    vmw = jnp.zeros((8, 128), f32).at[:3].set(p['vel_W'].astype(f32) @ nW[64:128]) \
                                  .at[3].set((p['mass_W'].astype(f32) @ nW[128:192])[0])
    ibias = (p['node_b'] + p['vel_b'] @ nW[64:128] + p['mass_b'] @ nW[128:192]).astype(f32)
    ibias8 = jnp.zeros((8, 128), f32).at[0].set(ibias)

    h = pl.pallas_call(
        _init_kernel, grid=(nrt,),
        in_specs=[_row_spec(1), _row_spec(8), _full_spec((104, 128)),
                  _full_spec((8, 128)), _full_spec((8, 128))],
        out_specs=_row_spec(128),
        out_shape=jax.ShapeDtypeStruct((NP, 128), f32),
    )(types, feat, embw, vmw, ibias8)

    x = x0
    xt3 = x0.reshape(nrt, NT, 8).transpose(0, 2, 1)
    x03 = xt3

    for lp in p['layers']:
        w1 = lp['edge_W1'].astype(f32)
        w1a, w1b, wdv = w1[:128], w1[128:256], w1[256]
        b1r = jnp.zeros((8, 128), f32).at[0].set(lp['edge_b1'].astype(f32))
        p1, p2 = pl.pallas_call(
            _proj_kernel, grid=(nrt,),
            in_specs=[_row_spec(128), _full_spec((128, 128)),
                      _full_spec((128, 128)), _full_spec((8, 128))],
            out_specs=[_row_spec(128), _row_spec(128)],
            out_shape=[jax.ShapeDtypeStruct((NP, 128), f32)] * 2,
            )(h, w1a, w1b, b1r)
        p23 = p2.reshape(nrt, NT, 128)

        cw2p = jnp.zeros((128, 8), f32).at[:, 0].set(lp['coord_W2'].astype(f32)[:, 0])
        vecs = jnp.stack([
            wdv, lp['edge_b2'], lp['coord_b1'],
            jnp.full((128,), lp['coord_b2'][0]),
            lp['node_b1'], lp['node_b2'], lp['ln_g'], lp['ln_b'],
        ]).astype(f32)

        grid_spec = pltpu.PrefetchScalarGridSpec(
            num_scalar_prefetch=2,
            grid=(nrt,),
            in_specs=[
                _row_spec(8), _row_spec(8), _row_spec(128), _row_spec(128),
                _row_spec(1),
                _full_spec((nrt, 8, NT)), _full_spec((nrt, 8, NT)),
                _full_spec((nrt, NT, 128)), _full_spec((nrt, 1, NT)),
                _full_spec((128, 128)), _full_spec((128, 128)),
                _full_spec((128, 8)), _full_spec((256, 128)),
                _full_spec((128, 128)), _full_spec((8, 128)),
            ],
            out_specs=[
                _row_spec(8),
                pl.BlockSpec((1, 8, NT), lambda r, *_: (r, 0, 0)),
                _row_spec(128),
            ],
        )
        x, xt3, h = pl.pallas_call(
            _layer_kernel, grid_spec=grid_spec,
            out_shape=[jax.ShapeDtypeStruct((NP, 8), f32),
                       jax.ShapeDtypeStruct((nrt, 8, NT), f32),
                       jax.ShapeDtypeStruct((NP, 128), f32)],
            compiler_params=pltpu.CompilerParams(
                dimension_semantics=("arbitrary",)),
            )(ct_start, ct_num, x0, x, h, p1, brow, x03, xt3, p23, b3,
          lp['edge_W2'].astype(f32), lp['coord_W1'].astype(f32), cw2p,
          lp['node_W1'].astype(f32), lp['node_W2'].astype(f32), vecs)

    hw = jnp.zeros((128, 16), f32).at[:, 0:3].set(p['head_pos_W'].astype(f32)) \
                                  .at[:, 8:11].set(p['head_vel_W'].astype(f32))
    hb = jnp.zeros((8, 16), f32).at[0, 0:3].set(p['head_pos_b'].astype(f32)) \
                                .at[0, 8:11].set(p['head_vel_b'].astype(f32)) \
                                .at[1, 0:8].set(MAXD).at[1, 8:16].set(MAXV)
    out = pl.pallas_call(
        _head_kernel, grid=(nrt,),
        in_specs=[_row_spec(128), _full_spec((128, 16)), _full_spec((8, 16))],
        out_specs=_row_spec(16),
        out_shape=jax.ShapeDtypeStruct((NP, 16), f32),
    )(h, hw, hb)
    return {'delta_pos': out[newpos, 0:3], 'delta_vel': out[newpos, 8:11]}


# speedup vs baseline: 6.9497x; 6.9497x over previous
import jax
import jax.numpy as jnp
from jax.experimental import pallas as pl
from jax.experimental.pallas import tpu as pltpu

NT = 128
CUT2 = 0.7 * 0.7
MAXD = 0.02
MAXV = 0.1

f32 = jnp.float32


def _dot(a, b):
    return jax.lax.dot_general(a, b, (((a.ndim - 1,), (0,)), ((), ())),
                               preferred_element_type=f32)


def _init_kernel(types_ref, feat_ref, embw_ref, vmw_ref, bias_ref, h0_ref):
    t = types_ref[...]
    nat = embw_ref.shape[0]
    onehot = (t == jax.lax.broadcasted_iota(jnp.int32, (NT, nat), 1)).astype(f32)
    h0 = _dot(onehot, embw_ref[...]) + _dot(feat_ref[...], vmw_ref[...])
    h0_ref[...] = h0 + bias_ref[0:1, :]


def _proj_kernel(h_ref, w1a_ref, w1b_ref, b1_ref, p1_ref, p2_ref):
    h = h_ref[...]
    p1_ref[...] = _dot(h, w1a_ref[...]) + b1_ref[0:1, :]
    p2_ref[...] = _dot(h, w1b_ref[...])


def _head_kernel(h_ref, hw_ref, hb_ref, out_ref):
    z = jnp.tanh(_dot(h_ref[...], hw_ref[...]) + hb_ref[0:1, :])
    out_ref[...] = z * hb_ref[1:2, :]


def _layer_kernel(cts_ref, ctn_ref,
                  x0r_ref, xr_ref, hr_ref, p1r_ref, br_ref,
                  xt0_ref, xt_ref, p2_ref, b3_ref,
                  w2_ref, cw1_ref, cw2_ref, nw1_ref, nw2_ref, vec_ref,
                  xout_ref, xtout_ref, hout_ref):
    r = pl.program_id(0)
    cstart = cts_ref[r]
    cnum = ctn_ref[r]
    x0i = x0r_ref[...]
    xi = xr_ref[...]
    hi = hr_ref[...]
    p1 = p1r_ref[...]
    bi = br_ref[...]
    wd = vec_ref[0:1, :]
    b2 = vec_ref[1:2, :]
    cb1 = vec_ref[2:3, :]
    cb2 = vec_ref[3:4, :]
    nb1 = vec_ref[4:5, :]
    nb2 = vec_ref[5:6, :]
    lng = vec_ref[6:7, :]
    lnb = vec_ref[7:8, :]
    w2 = w2_ref[...].astype(jnp.bfloat16)
    cw1 = cw1_ref[...].astype(jnp.bfloat16)
    cw2 = cw2_ref[...].astype(jnp.bfloat16)
    p1b = p1.astype(jnp.bfloat16)
    b2b = b2.astype(jnp.bfloat16)
    cb1b = cb1.astype(jnp.bfloat16)
    wdb = wd.astype(jnp.bfloat16)

    row_ids = r * NT + jax.lax.broadcasted_iota(jnp.int32, (NT, 1), 0)

    def body(c, carry):
        agg, s, t, cnt = carry
        ct = cstart + c
        xt0j = xt0_ref[ct]
        xtj = xt_ref[ct]
        p2j = p2_ref[ct]
        bj = b3_ref[ct]

        d0 = jnp.zeros((NT, NT), f32)
        d2 = jnp.zeros((NT, NT), f32)
        for d in range(3):
            f0 = x0i[:, d:d + 1] - xt0j[d:d + 1, :]
            d0 = d0 + f0 * f0
            f1 = xi[:, d:d + 1] - xtj[d:d + 1, :]
            d2 = d2 + f1 * f1
        col_ids = ct * NT + jax.lax.broadcasted_iota(jnp.int32, (1, NT), 1)
        mask = (d0 <= CUT2) & (bi == bj) & (row_ids != col_ids)
        mf = mask.astype(f32)

        bf = jnp.bfloat16
        d2b = d2.astype(bf)
        e3 = (p1b[:, None, :] + p2j.astype(bf)[None, :, :]
              + d2b[:, :, None] * wdb[None, :, :])
        ef1 = jax.nn.silu(e3).reshape(NT * NT, 128)
        ef2 = jax.nn.silu(_dot(ef1, w2).astype(bf) + b2b)
        cg1 = jax.nn.silu(_dot(ef2, cw1).astype(bf) + cb1b)
        g8 = _dot(cg1, cw2)
        gate = g8.reshape(NT, NT, 8).sum(axis=2) + cb2
        gm = gate * mf
        ef23 = ef2.reshape(NT, NT, 128)
        mf3 = mf.reshape(NT, 1, NT).astype(jnp.bfloat16)
        aggc = jax.lax.dot_general(
            mf3, ef23, (((2,), (1,)), ((0,), (0,))),
            preferred_element_type=f32)
        agg = agg + aggc.reshape(NT, 128)
        s = s + gm.sum(axis=1, keepdims=True)
        t = t + jax.lax.dot_general(gm, xtj, (((1,), (1,)), ((), ())),
                                    preferred_element_type=f32)
        cnt = cnt + mf.sum(axis=1, keepdims=True)
        return agg, s, t, cnt

    agg0 = (jnp.zeros((NT, 128), f32), jnp.zeros((NT, 1), f32),
            jnp.zeros((NT, 8), f32), jnp.zeros((NT, 1), f32))
    agg, s, t, cnt = jax.lax.fori_loop(0, cnum, body, agg0)

    cntc = jnp.maximum(cnt, 1.0)
    xn = xi + (xi * s - t) / cntc
    aggm = agg / cntc
    node_in = jnp.concatenate([hi, aggm], axis=1)
    d1 = jax.nn.silu(_dot(node_in, nw1_ref[...]) + nb1)
    hn = hi + _dot(d1, nw2_ref[...]) + nb2
    mu = jnp.mean(hn, axis=1, keepdims=True)
    var = jnp.mean((hn - mu) ** 2, axis=1, keepdims=True)
    hln = (hn - mu) * jax.lax.rsqrt(var + 1e-5) * lng + lnb
    hout_ref[...] = jax.nn.silu(hln)
    xout_ref[...] = xn
    xtout_ref[0] = xn.T


def _row_spec(w):
    return pl.BlockSpec((NT, w), lambda r, *_: (r, 0))


def _full_spec(shape):
    n = len(shape)
    return pl.BlockSpec(shape, lambda r, *_, _n=n: (0,) * _n)


def kernel(x_t, v_t, atom_types, masses, batch, params):
    N = x_t.shape[0]
    NG = 20
    i32 = jnp.int32
    nrt = N // NT + NG + 1
    NP = nrt * NT
    p = params

    b = batch.astype(i32)
    gidx = jnp.arange(NG, dtype=i32)
    gstart = jnp.searchsorted(b, gidx, side='left').astype(i32)
    gend = jnp.searchsorted(b, gidx, side='right').astype(i32)
    sizes = gend - gstart
    tcnt = (sizes + NT - 1) // NT
    prows = tcnt * NT
    pstart = (jnp.cumsum(prows) - prows).astype(i32)
    total_tiles = jnp.sum(tcnt)
    newpos = pstart[b] + (jnp.arange(N, dtype=i32) - gstart[b])

    arp = jnp.arange(NP, dtype=i32)
    gp = jnp.clip(jnp.searchsorted(pstart, arp, side='right').astype(i32) - 1,
                  0, NG - 1)
    off = arp - pstart[gp]
    oldidx = jnp.where(off < sizes[gp], gstart[gp] + off, N)

    xe = jnp.concatenate([x_t.astype(f32), jnp.zeros((1, 3), f32)])[oldidx]
    x0 = jnp.concatenate([xe, jnp.zeros((NP, 5), f32)], axis=1)
    ve = jnp.concatenate([v_t.astype(f32), jnp.zeros((1, 3), f32)])[oldidx]
    me = jnp.concatenate([masses.astype(f32), jnp.zeros((1,), f32)])[oldidx]
    feat = jnp.concatenate([ve, me[:, None], jnp.zeros((NP, 4), f32)], axis=1)
    types = jnp.concatenate([atom_types.astype(i32),
                             jnp.full((1,), -1, i32)])[oldidx][:, None]
    bcol = jnp.concatenate([b, jnp.full((1,), -1, i32)])[oldidx]
    brow = bcol.reshape(NP, 1)
    b3 = bcol.reshape(nrt, 1, NT)

    ar = jnp.arange(nrt, dtype=i32)
    tstartg = pstart // NT
    tg = jnp.clip(jnp.searchsorted(tstartg, ar, side='right').astype(i32) - 1,
                  0, NG - 1)
    ct_start = tstartg[tg]
    ct_num = jnp.where(ar < total_tiles, tcnt[tg], 0).astype(i32)

    nW = p['node_W'].astype(f32)
    embw = jnp.zeros((104, 128), f32).at[:100].set(p['atom_emb'].astype(f32) @ nW[:64])
    vmw = jnp.zeros((8, 128), f32).at[:3].set(p['vel_W'].astype(f32) @ nW[64:128]) \
                                  .at[3].set((p['mass_W'].astype(f32) @ nW[128:192])[0])
    ibias = (p['node_b'] + p['vel_b'] @ nW[64:128] + p['mass_b'] @ nW[128:192]).astype(f32)
    ibias8 = jnp.zeros((8, 128), f32).at[0].set(ibias)

    h = pl.pallas_call(
        _init_kernel, grid=(nrt,),
        in_specs=[_row_spec(1), _row_spec(8), _full_spec((104, 128)),
                  _full_spec((8, 128)), _full_spec((8, 128))],
        out_specs=_row_spec(128),
        out_shape=jax.ShapeDtypeStruct((NP, 128), f32),
    )(types, feat, embw, vmw, ibias8)

    x = x0
    xt3 = x0.reshape(nrt, NT, 8).transpose(0, 2, 1)
    x03 = xt3

    for lp in p['layers']:
        w1 = lp['edge_W1'].astype(f32)
        w1a, w1b, wdv = w1[:128], w1[128:256], w1[256]
        b1r = jnp.zeros((8, 128), f32).at[0].set(lp['edge_b1'].astype(f32))
        p1, p2 = pl.pallas_call(
            _proj_kernel, grid=(nrt,),
            in_specs=[_row_spec(128), _full_spec((128, 128)),
                      _full_spec((128, 128)), _full_spec((8, 128))],
            out_specs=[_row_spec(128), _row_spec(128)],
            out_shape=[jax.ShapeDtypeStruct((NP, 128), f32)] * 2,
            )(h, w1a, w1b, b1r)
        p23 = p2.reshape(nrt, NT, 128)

        cw2p = jnp.zeros((128, 8), f32).at[:, 0].set(lp['coord_W2'].astype(f32)[:, 0])
        vecs = jnp.stack([
            wdv, lp['edge_b2'], lp['coord_b1'],
            jnp.full((128,), lp['coord_b2'][0]),
            lp['node_b1'], lp['node_b2'], lp['ln_g'], lp['ln_b'],
        ]).astype(f32)

        grid_spec = pltpu.PrefetchScalarGridSpec(
            num_scalar_prefetch=2,
            grid=(nrt,),
            in_specs=[
                _row_spec(8), _row_spec(8), _row_spec(128), _row_spec(128),
                _row_spec(1),
                _full_spec((nrt, 8, NT)), _full_spec((nrt, 8, NT)),
                _full_spec((nrt, NT, 128)), _full_spec((nrt, 1, NT)),
                _full_spec((128, 128)), _full_spec((128, 128)),
                _full_spec((128, 8)), _full_spec((256, 128)),
                _full_spec((128, 128)), _full_spec((8, 128)),
            ],
            out_specs=[
                _row_spec(8),
                pl.BlockSpec((1, 8, NT), lambda r, *_: (r, 0, 0)),
                _row_spec(128),
            ],
        )
        x, xt3, h = pl.pallas_call(
            _layer_kernel, grid_spec=grid_spec,
            out_shape=[jax.ShapeDtypeStruct((NP, 8), f32),
                       jax.ShapeDtypeStruct((nrt, 8, NT), f32),
                       jax.ShapeDtypeStruct((NP, 128), f32)],
            compiler_params=pltpu.CompilerParams(
                dimension_semantics=("arbitrary",)),
            )(ct_start, ct_num, x0, x, h, p1, brow, x03, xt3, p23, b3,
          lp['edge_W2'].astype(f32), lp['coord_W1'].astype(f32), cw2p,
          lp['node_W1'].astype(f32), lp['node_W2'].astype(f32), vecs)

    hw = jnp.zeros((128, 16), f32).at[:, 0:3].set(p['head_pos_W'].astype(f32)) \
                                  .at[:, 8:11].set(p['head_vel_W'].astype(f32))
    hb = jnp.zeros((8, 16), f32).at[0, 0:3].set(p['head_pos_b'].astype(f32)) \
                                .at[0, 8:11].set(p['head_vel_b'].astype(f32)) \
                                .at[1, 0:8].set(MAXD).at[1, 8:16].set(MAXV)
    out = pl.pallas_call(
        _head_kernel, grid=(nrt,),
        in_specs=[_row_spec(128), _full_spec((128, 16)), _full_spec((8, 16))],
        out_specs=_row_spec(16),
        out_shape=jax.ShapeDtypeStruct((NP, 16), f32),
    )(h, hw, hb)
    return {'delta_pos': out[newpos, 0:3], 'delta_vel': out[newpos, 8:11]}
